# Initial kernel scaffold; baseline (speedup 1.0000x reference)
import functools

import jax
import jax.numpy as jnp
from jax import lax
from jax.experimental import pallas as pl
from jax.experimental.pallas import tpu as pltpu
from jax.experimental.pallas import tpu_sc as plsc

N = 10000
DF = 128
H = 128
OUT = 64
E = 320000
E2 = 2 * E

NC = 2
NS = 16
NW = NC * NS
L = 16

NP = 10240
STRIPE = NP // NS
PAD_NODE = NP - 1

CB = 128
KCH = -(-(E // NW) // CB)
KB2 = -(-(E2 // NW) // CB)

BN = 1024

_mesh = plsc.VectorSubcoreMesh(core_axis_name="c", subcore_axis_name="s")


@functools.partial(
    pl.kernel,
    out_type=jax.ShapeDtypeStruct((NW, NP), jnp.float32),
    mesh=_mesh,
    scratch_types=[
        pltpu.VMEM((KCH, CB), jnp.int32),
        pltpu.VMEM((NP,), jnp.float32),
    ],
)
def _deg_kernel(dst_hbm, degp_hbm, idx_v, deg_v):
    c = lax.axis_index("c")
    s = lax.axis_index("s")
    wid = s * NC + c
    pltpu.sync_copy(dst_hbm.at[wid], idx_v)
    zero = jnp.zeros((L,), jnp.float32)

    def zbody(i, carry):
        deg_v[pl.ds(i * L, L)] = zero
        return carry

    lax.fori_loop(0, NP // L, zbody, 0)
    ones = jnp.ones((L,), jnp.float32)

    def cbody(j, carry):
        for g in range(CB // L):
            idx16 = idx_v[j, pl.ds(g * L, L)]
            plsc.addupdate_scatter(deg_v, [idx16], ones)
        return carry

    lax.fori_loop(0, KCH, cbody, 0)
    pltpu.sync_copy(deg_v, degp_hbm.at[wid])


def _make_agg(dim):
    @functools.partial(
        pl.kernel,
        out_type=jax.ShapeDtypeStruct((NC, NP, dim), jnp.float32),
        mesh=_mesh,
        scratch_types=[
            pltpu.VMEM((KCH, CB), jnp.int32),
            pltpu.VMEM((KCH, CB), jnp.int32),
            pltpu.VMEM((CB, dim), jnp.float32),
            pltpu.VMEM_SHARED((NP, dim), jnp.float32),
            pltpu.SemaphoreType.DMA,
        ],
        name=f"edge_agg_{dim}",
    )
    def agg(table_hbm, src_hbm, dst_hbm, zeros_hbm, out_hbm, src_v, dst_v,
            rows, shared, sem):
        c = lax.axis_index("c")
        s = lax.axis_index("s")
        wid = s * NC + c
        stripe = pl.ds(s * STRIPE, STRIPE)
        pltpu.sync_copy(zeros_hbm, shared.at[stripe])
        pltpu.sync_copy(src_hbm.at[wid], src_v)
        pltpu.sync_copy(dst_hbm.at[wid], dst_v)
        plsc.subcore_barrier()

        def body(j, carry):
            pltpu.async_copy(table_hbm.at[src_v.at[j]], rows, sem).wait()
            pltpu.sync_copy(rows, shared.at[dst_v.at[j]], add=True)
            return carry

        lax.fori_loop(0, KCH, body, 0)
        plsc.subcore_barrier()
        pltpu.sync_copy(shared.at[stripe], out_hbm.at[c, stripe])

    return agg


_agg_h = _make_agg(H)
_agg_out = _make_agg(OUT)


@functools.partial(
    pl.kernel,
    out_type=jax.ShapeDtypeStruct((NW, KB2, CB), jnp.float32),
    mesh=_mesh,
    scratch_types=[
        pltpu.VMEM((KB2, CB), jnp.int32),
        pltpu.VMEM((KB2, CB), jnp.int32),
        pltpu.VMEM((CB, OUT), jnp.float32),
        pltpu.VMEM((CB, OUT), jnp.float32),
        pltpu.VMEM((CB,), jnp.float32),
        pltpu.SemaphoreType.DMA,
        pltpu.SemaphoreType.DMA,
    ],
    name="edge_decoder",
)
def _dec_kernel(latent_hbm, ia_hbm, ib_hbm, out_hbm, ia_v, ib_v, ra, rb, lbuf,
                sema, semb):
    c = lax.axis_index("c")
    s = lax.axis_index("s")
    wid = s * NC + c
    pltpu.sync_copy(ia_hbm.at[wid], ia_v)
    pltpu.sync_copy(ib_hbm.at[wid], ib_v)
    iota = lax.iota(jnp.int32, L)

    def chunk(j, carry):
        ca = pltpu.async_copy(latent_hbm.at[ia_v.at[j]], ra, sema)
        cb = pltpu.async_copy(latent_hbm.at[ib_v.at[j]], rb, semb)
        ca.wait()
        cb.wait()

        def group(g, gcarry):
            rowi = g * L + iota
            acc = jnp.zeros((L,), jnp.float32)
            for d in range(OUT):
                cold = jnp.full((L,), d, jnp.int32)
                va = plsc.load_gather(ra, [rowi, cold])
                vb = plsc.load_gather(rb, [rowi, cold])
                acc = acc + va * vb
            lbuf[pl.ds(g * L, L)] = acc
            return gcarry

        lax.fori_loop(0, CB // L, group, 0)
        pltpu.sync_copy(lbuf, out_hbm.at[wid, j])
        return carry

    lax.fori_loop(0, KB2, chunk, 0)


def _dinv_block(degp_blk, block_i):
    deg = jnp.sum(degp_blk, axis=0)[:, None] + 1.0
    rows = lax.broadcasted_iota(jnp.int32, (BN, 1), 0) + block_i * BN
    return jnp.where(rows < N, lax.rsqrt(deg), 0.0)


def _tc1_body(x_ref, w_ref, degp_ref, h1p_ref):
    i = pl.program_id(0)
    dinv = _dinv_block(degp_ref[...], i)
    h = jnp.dot(x_ref[...], w_ref[...], preferred_element_type=jnp.float32)
    h1p_ref[...] = h * dinv


def _tc2_body(p_ref, h1p_ref, degp_ref, w2_ref, b1_ref, h2p_ref):
    i = pl.program_id(0)
    dinv = _dinv_block(degp_ref[...], i)
    agg = p_ref[0] + p_ref[1] + h1p_ref[...]
    z1 = jnp.maximum(dinv * agg + b1_ref[...], 0.0)
    h2p_ref[...] = jnp.dot(z1, w2_ref[...], preferred_element_type=jnp.float32) * dinv


def _tc3_body(q_ref, h2p_ref, degp_ref, b2_ref, lat_ref):
    i = pl.program_id(0)
    dinv = _dinv_block(degp_ref[...], i)
    lat_ref[...] = dinv * (q_ref[0] + q_ref[1] + h2p_ref[...]) + b2_ref[...]


def _tc1(x_p, W1, degp):
    return pl.pallas_call(
        _tc1_body,
        grid=(NP // BN,),
        in_specs=[
            pl.BlockSpec((BN, DF), lambda i: (i, 0)),
            pl.BlockSpec((DF, H), lambda i: (0, 0)),
            pl.BlockSpec((NW, BN), lambda i: (0, i)),
        ],
        out_specs=pl.BlockSpec((BN, H), lambda i: (i, 0)),
        out_shape=jax.ShapeDtypeStruct((NP, H), jnp.float32),
    )(x_p, W1, degp)


def _tc2(p, h1p, degp, W2, b1):
    return pl.pallas_call(
        _tc2_body,
        grid=(NP // BN,),
        in_specs=[
            pl.BlockSpec((NC, BN, H), lambda i: (0, i, 0)),
            pl.BlockSpec((BN, H), lambda i: (i, 0)),
            pl.BlockSpec((NW, BN), lambda i: (0, i)),
            pl.BlockSpec((H, OUT), lambda i: (0, 0)),
            pl.BlockSpec((1, H), lambda i: (0, 0)),
        ],
        out_specs=pl.BlockSpec((BN, OUT), lambda i: (i, 0)),
        out_shape=jax.ShapeDtypeStruct((NP, OUT), jnp.float32),
    )(p, h1p, degp, W2, b1)


def _tc3(q, h2p, degp, b2):
    return pl.pallas_call(
        _tc3_body,
        grid=(NP // BN,),
        in_specs=[
            pl.BlockSpec((NC, BN, OUT), lambda i: (0, i, 0)),
            pl.BlockSpec((BN, OUT), lambda i: (i, 0)),
            pl.BlockSpec((NW, BN), lambda i: (0, i)),
            pl.BlockSpec((1, OUT), lambda i: (0, 0)),
        ],
        out_specs=pl.BlockSpec((BN, OUT), lambda i: (i, 0)),
        out_shape=jax.ShapeDtypeStruct((NP, OUT), jnp.float32),
    )(q, h2p, degp, b2)


def _pad_idx(idx, kch):
    tot = NW * kch * CB
    p = jnp.full((tot,), PAD_NODE, jnp.int32).at[: idx.shape[0]].set(idx)
    return p.reshape(NW, kch, CB)


def kernel(x, edge_index, neg_edge, W1, b1, W2, b2):
    src = _pad_idx(edge_index[0], KCH)
    dst = _pad_idx(edge_index[1], KCH)

    x_p = jnp.zeros((NP, DF), jnp.float32).at[:N].set(x)
    zeros_h = jnp.zeros((STRIPE, H), jnp.float32)
    zeros_o = jnp.zeros((STRIPE, OUT), jnp.float32)

    degp = _deg_kernel(dst)
    h1p = _tc1(x_p, W1, degp)
    p1 = _agg_h(h1p, src, dst, zeros_h)
    h2p = _tc2(p1, h1p, degp, W2, b1.reshape(1, H))
    p2 = _agg_out(h2p, src, dst, zeros_o)
    latent = _tc3(p2, h2p, degp, b2.reshape(1, OUT))

    ia = _pad_idx(jnp.concatenate([edge_index[0], neg_edge[0]]), KB2)
    ib = _pad_idx(jnp.concatenate([edge_index[1], neg_edge[1]]), KB2)
    logits = _dec_kernel(latent, ia, ib)
    return logits.reshape(-1)[:E2]


# speedup vs baseline: 6.4134x; 6.4134x over previous
import functools

import jax
import jax.numpy as jnp
from jax import lax
from jax.experimental import pallas as pl
from jax.experimental.pallas import tpu as pltpu
from jax.experimental.pallas import tpu_sc as plsc

N = 10000
DF = 128
H = 128
OUT = 64
E = 320000
E2 = 2 * E

NC = 2
NS = 16
NW = NC * NS
L = 16

NP = 10240
STRIPE = NP // NS
PAD_NODE = NP - 1

CB = 128
KCH = -(-(E // NW) // CB)
KB2 = -(-(E2 // NW) // CB)

BN = 1024

_mesh = plsc.VectorSubcoreMesh(core_axis_name="c", subcore_axis_name="s")
_sc_params = pltpu.CompilerParams(
    needs_layout_passes=False, use_tc_tiling_on_sc=False
)


@functools.partial(
    pl.kernel,
    out_type=jax.ShapeDtypeStruct((NW, NP), jnp.float32),
    mesh=_mesh,
    scratch_types=[
        pltpu.VMEM((KCH, CB), jnp.int32),
        pltpu.VMEM((NP,), jnp.float32),
    ],
    compiler_params=_sc_params,
)
def _deg_kernel(dst_hbm, degp_hbm, idx_v, deg_v):
    c = lax.axis_index("c")
    s = lax.axis_index("s")
    wid = s * NC + c
    pltpu.sync_copy(dst_hbm.at[wid], idx_v)
    zero = jnp.zeros((L,), jnp.float32)

    def zbody(i, carry):
        deg_v[pl.ds(i * L, L)] = zero
        return carry

    lax.fori_loop(0, NP // L, zbody, 0)
    ones = jnp.ones((L,), jnp.float32)

    def cbody(j, carry):
        for g in range(CB // L):
            idx16 = idx_v[j, pl.ds(g * L, L)]
            plsc.addupdate_scatter(deg_v, [idx16], ones)
        return carry

    lax.fori_loop(0, KCH, cbody, 0)
    pltpu.sync_copy(deg_v, degp_hbm.at[wid])


def _make_agg(dim):
    @functools.partial(
        pl.kernel,
        out_type=jax.ShapeDtypeStruct((NC, NP, dim), jnp.float32),
        mesh=_mesh,
        scratch_types=[
            pltpu.VMEM((KCH, CB), jnp.int32),
            pltpu.VMEM((KCH, CB), jnp.int32),
            pltpu.VMEM((CB, dim), jnp.float32),
            pltpu.VMEM_SHARED((NP, dim), jnp.float32),
            pltpu.SemaphoreType.DMA,
        ],
        name=f"edge_agg_{dim}",
        compiler_params=_sc_params,
    )
    def agg(table_hbm, src_hbm, dst_hbm, zeros_hbm, out_hbm, src_v, dst_v,
            rows, shared, sem):
        c = lax.axis_index("c")
        s = lax.axis_index("s")
        wid = s * NC + c
        stripe = pl.ds(s * STRIPE, STRIPE)
        pltpu.sync_copy(zeros_hbm, shared.at[stripe])
        pltpu.sync_copy(src_hbm.at[wid], src_v)
        pltpu.sync_copy(dst_hbm.at[wid], dst_v)
        plsc.subcore_barrier()

        def body(j, carry):
            pltpu.async_copy(table_hbm.at[src_v.at[j]], rows, sem).wait()
            pltpu.sync_copy(rows, shared.at[dst_v.at[j]], add=True)
            return carry

        lax.fori_loop(0, KCH, body, 0)
        plsc.subcore_barrier()
        pltpu.sync_copy(shared.at[stripe], out_hbm.at[c, stripe])

    return agg


_agg_h = _make_agg(H)
_agg_out = _make_agg(OUT)


@functools.partial(
    pl.kernel,
    out_type=jax.ShapeDtypeStruct((NW, KB2, CB), jnp.float32),
    mesh=_mesh,
    scratch_types=[
        pltpu.VMEM((KB2, CB), jnp.int32),
        pltpu.VMEM((KB2, CB), jnp.int32),
        pltpu.VMEM((CB, OUT), jnp.float32),
        pltpu.VMEM((CB, OUT), jnp.float32),
        pltpu.VMEM((CB,), jnp.float32),
        pltpu.SemaphoreType.DMA,
        pltpu.SemaphoreType.DMA,
    ],
    name="edge_decoder",
    compiler_params=_sc_params,
)
def _dec_kernel(latent_hbm, ia_hbm, ib_hbm, out_hbm, ia_v, ib_v, ra, rb, lbuf,
                sema, semb):
    c = lax.axis_index("c")
    s = lax.axis_index("s")
    wid = s * NC + c
    pltpu.sync_copy(ia_hbm.at[wid], ia_v)
    pltpu.sync_copy(ib_hbm.at[wid], ib_v)
    iota = lax.iota(jnp.int32, L)

    def chunk(j, carry):
        ca = pltpu.async_copy(latent_hbm.at[ia_v.at[j]], ra, sema)
        cb = pltpu.async_copy(latent_hbm.at[ib_v.at[j]], rb, semb)
        ca.wait()
        cb.wait()

        def group(g, gcarry):
            rowi = g * L + iota
            acc = jnp.zeros((L,), jnp.float32)
            for d in range(OUT):
                cold = jnp.full((L,), d, jnp.int32)
                va = plsc.load_gather(ra, [rowi, cold])
                vb = plsc.load_gather(rb, [rowi, cold])
                acc = acc + va * vb
            lbuf[pl.ds(g * L, L)] = acc
            return gcarry

        lax.fori_loop(0, CB // L, group, 0)
        pltpu.sync_copy(lbuf, out_hbm.at[wid, j])
        return carry

    lax.fori_loop(0, KB2, chunk, 0)


def _dinv_block(degp_blk, block_i):
    deg = jnp.sum(degp_blk, axis=0)[:, None] + 1.0
    rows = lax.broadcasted_iota(jnp.int32, (BN, 1), 0) + block_i * BN
    return jnp.where(rows < N, lax.rsqrt(deg), 0.0)


def _tc1_body(x_ref, w_ref, degp_ref, h1p_ref):
    i = pl.program_id(0)
    dinv = _dinv_block(degp_ref[...], i)
    h = jnp.dot(x_ref[...], w_ref[...], preferred_element_type=jnp.float32)
    h1p_ref[...] = h * dinv


def _tc2_body(p_ref, h1p_ref, degp_ref, w2_ref, b1_ref, h2p_ref):
    i = pl.program_id(0)
    dinv = _dinv_block(degp_ref[...], i)
    agg = p_ref[0] + p_ref[1] + h1p_ref[...]
    z1 = jnp.maximum(dinv * agg + b1_ref[...], 0.0)
    h2p_ref[...] = jnp.dot(z1, w2_ref[...], preferred_element_type=jnp.float32) * dinv


def _tc3_body(q_ref, h2p_ref, degp_ref, b2_ref, lat_ref):
    i = pl.program_id(0)
    dinv = _dinv_block(degp_ref[...], i)
    lat_ref[...] = dinv * (q_ref[0] + q_ref[1] + h2p_ref[...]) + b2_ref[...]


def _tc1(x_p, W1, degp):
    return pl.pallas_call(
        _tc1_body,
        grid=(NP // BN,),
        in_specs=[
            pl.BlockSpec((BN, DF), lambda i: (i, 0)),
            pl.BlockSpec((DF, H), lambda i: (0, 0)),
            pl.BlockSpec((NW, BN), lambda i: (0, i)),
        ],
        out_specs=pl.BlockSpec((BN, H), lambda i: (i, 0)),
        out_shape=jax.ShapeDtypeStruct((NP, H), jnp.float32),
    )(x_p, W1, degp)


def _tc2(p, h1p, degp, W2, b1):
    return pl.pallas_call(
        _tc2_body,
        grid=(NP // BN,),
        in_specs=[
            pl.BlockSpec((NC, BN, H), lambda i: (0, i, 0)),
            pl.BlockSpec((BN, H), lambda i: (i, 0)),
            pl.BlockSpec((NW, BN), lambda i: (0, i)),
            pl.BlockSpec((H, OUT), lambda i: (0, 0)),
            pl.BlockSpec((1, H), lambda i: (0, 0)),
        ],
        out_specs=pl.BlockSpec((BN, OUT), lambda i: (i, 0)),
        out_shape=jax.ShapeDtypeStruct((NP, OUT), jnp.float32),
    )(p, h1p, degp, W2, b1)


def _tc3(q, h2p, degp, b2):
    return pl.pallas_call(
        _tc3_body,
        grid=(NP // BN,),
        in_specs=[
            pl.BlockSpec((NC, BN, OUT), lambda i: (0, i, 0)),
            pl.BlockSpec((BN, OUT), lambda i: (i, 0)),
            pl.BlockSpec((NW, BN), lambda i: (0, i)),
            pl.BlockSpec((1, OUT), lambda i: (0, 0)),
        ],
        out_specs=pl.BlockSpec((BN, OUT), lambda i: (i, 0)),
        out_shape=jax.ShapeDtypeStruct((NP, OUT), jnp.float32),
    )(q, h2p, degp, b2)


def _pad_idx(idx, kch):
    tot = NW * kch * CB
    p = jnp.full((tot,), PAD_NODE, jnp.int32).at[: idx.shape[0]].set(idx)
    return p.reshape(NW, kch, CB)


def kernel(x, edge_index, neg_edge, W1, b1, W2, b2):
    src = _pad_idx(edge_index[0], KCH)
    dst = _pad_idx(edge_index[1], KCH)

    x_p = jnp.zeros((NP, DF), jnp.float32).at[:N].set(x)
    zeros_h = jnp.zeros((STRIPE, H), jnp.float32)
    zeros_o = jnp.zeros((STRIPE, OUT), jnp.float32)

    degp = _deg_kernel(dst)
    h1p = _tc1(x_p, W1, degp)
    p1 = _agg_h(h1p, src, dst, zeros_h)
    h2p = _tc2(p1, h1p, degp, W2, b1.reshape(1, H))
    p2 = _agg_out(h2p, src, dst, zeros_o)
    latent = _tc3(p2, h2p, degp, b2.reshape(1, OUT))

    ia = _pad_idx(jnp.concatenate([edge_index[0], neg_edge[0]]), KB2)
    ib = _pad_idx(jnp.concatenate([edge_index[1], neg_edge[1]]), KB2)
    logits = _dec_kernel(latent, ia, ib)
    return logits.reshape(-1)[:E2]


# speedup vs baseline: 6.8188x; 1.0632x over previous
import functools

import jax
import jax.numpy as jnp
from jax import lax
from jax.experimental import pallas as pl
from jax.experimental.pallas import tpu as pltpu
from jax.experimental.pallas import tpu_sc as plsc

N = 10000
DF = 128
H = 128
OUT = 64
E = 320000
E2 = 2 * E

NC = 2
NS = 16
NW = NC * NS
L = 16

NP = 10240
STRIPE = NP // NS
PAD_NODE = NP - 1

CB = 128
KCH = 80
CB1 = 64
KCH1 = 160
KB2 = 158

BN = 1024

_mesh = plsc.VectorSubcoreMesh(core_axis_name="c", subcore_axis_name="s")
_sc_params = pltpu.CompilerParams(
    needs_layout_passes=False, use_tc_tiling_on_sc=False
)


@functools.partial(
    pl.kernel,
    out_type=jax.ShapeDtypeStruct((NW, NP), jnp.float32),
    mesh=_mesh,
    scratch_types=[
        pltpu.VMEM((KCH, CB), jnp.int32),
        pltpu.VMEM((NP,), jnp.float32),
    ],
    compiler_params=_sc_params,
)
def _deg_kernel(dst_hbm, degp_hbm, idx_v, deg_v):
    c = lax.axis_index("c")
    s = lax.axis_index("s")
    wid = s * NC + c
    pltpu.sync_copy(dst_hbm.at[wid], idx_v)
    zero = jnp.zeros((L,), jnp.float32)

    def zbody(i, carry):
        deg_v[pl.ds(i * L, L)] = zero
        return carry

    lax.fori_loop(0, NP // L, zbody, 0)
    ones = jnp.ones((L,), jnp.float32)

    def cbody(j, carry):
        for g in range(CB // L):
            idx16 = idx_v[j, pl.ds(g * L, L)]
            plsc.addupdate_scatter(deg_v, [idx16], ones)
        return carry

    lax.fori_loop(0, KCH, cbody, 0)
    pltpu.sync_copy(deg_v, degp_hbm.at[wid])


def _make_agg(dim, kch, cb):
    @functools.partial(
        pl.kernel,
        out_type=jax.ShapeDtypeStruct((NC, NP, dim), jnp.float32),
        mesh=_mesh,
        scratch_types=[
            pltpu.VMEM((kch, cb), jnp.int32),
            pltpu.VMEM((kch, cb), jnp.int32),
            pltpu.VMEM((cb, dim), jnp.float32),
            pltpu.VMEM((cb, dim), jnp.float32),
            pltpu.VMEM_SHARED((NP, dim), jnp.float32),
            pltpu.SemaphoreType.DMA,
            pltpu.SemaphoreType.DMA,
        ],
        name=f"edge_agg_{dim}",
        compiler_params=_sc_params,
    )
    def agg(table_hbm, src_hbm, dst_hbm, zeros_hbm, out_hbm, src_v, dst_v,
            rows0, rows1, shared, sem0, sem1):
        c = lax.axis_index("c")
        s = lax.axis_index("s")
        wid = s * NC + c
        stripe = pl.ds(s * STRIPE, STRIPE)
        pltpu.sync_copy(zeros_hbm, shared.at[stripe])
        pltpu.sync_copy(src_hbm.at[wid], src_v)
        pltpu.sync_copy(dst_hbm.at[wid], dst_v)
        plsc.subcore_barrier()

        def gather(j, rows, sem):
            pltpu.async_copy(table_hbm.at[src_v.at[j]], rows, sem)

        def gwait(j, rows, sem):
            pltpu.make_async_copy(table_hbm.at[src_v.at[j]], rows, sem).wait()

        gather(0, rows0, sem0)

        def body(j, carry):
            gwait(j, rows0, sem0)
            gather(j + 1, rows1, sem1)
            pltpu.sync_copy(rows0, shared.at[dst_v.at[j]], add=True)
            gwait(j + 1, rows1, sem1)

            @pl.when(j + 2 < kch)
            def _():
                gather(j + 2, rows0, sem0)

            pltpu.sync_copy(rows1, shared.at[dst_v.at[j + 1]], add=True)
            return carry

        lax.fori_loop(0, kch // 2, lambda t, cc: body(t * 2, cc), 0)
        plsc.subcore_barrier()
        pltpu.sync_copy(shared.at[stripe], out_hbm.at[c, stripe])

    return agg


_agg_h = _make_agg(H, KCH1, CB1)
_agg_out = _make_agg(OUT, KCH, CB)


@functools.partial(
    pl.kernel,
    out_type=jax.ShapeDtypeStruct((NW, KB2 * CB), jnp.float32),
    mesh=_mesh,
    scratch_types=[
        pltpu.VMEM((KB2, CB), jnp.int32),
        pltpu.VMEM((KB2, CB), jnp.int32),
        pltpu.VMEM((CB, OUT), jnp.float32),
        pltpu.VMEM((CB, OUT), jnp.float32),
        pltpu.VMEM((CB, OUT), jnp.float32),
        pltpu.VMEM((CB, OUT), jnp.float32),
        pltpu.VMEM((KB2 * CB,), jnp.float32),
        pltpu.SemaphoreType.DMA,
        pltpu.SemaphoreType.DMA,
    ],
    name="edge_decoder",
    compiler_params=_sc_params,
)
def _dec_kernel(latent_hbm, ia_hbm, ib_hbm, out_hbm, ia_v, ib_v,
                ra0, rb0, ra1, rb1, lbuf, sem0, sem1):
    c = lax.axis_index("c")
    s = lax.axis_index("s")
    wid = s * NC + c
    pltpu.sync_copy(ia_hbm.at[wid], ia_v)
    pltpu.sync_copy(ib_hbm.at[wid], ib_v)
    iota = lax.iota(jnp.int32, L)

    def gather(j, ra, rb, sem):
        pltpu.async_copy(latent_hbm.at[ia_v.at[j]], ra, sem)
        pltpu.async_copy(latent_hbm.at[ib_v.at[j]], rb, sem)

    def gwait(j, ra, rb, sem):
        pltpu.make_async_copy(latent_hbm.at[ia_v.at[j]], ra, sem).wait()
        pltpu.make_async_copy(latent_hbm.at[ib_v.at[j]], rb, sem).wait()

    def compute(j, ra, rb):
        def group(g, gcarry):
            rowi = g * L + iota
            a0 = jnp.zeros((L,), jnp.float32)
            a1 = a0
            a2 = a0
            a3 = a0
            for d in range(0, OUT, 4):
                c0 = jnp.full((L,), d, jnp.int32)
                c1 = jnp.full((L,), d + 1, jnp.int32)
                c2 = jnp.full((L,), d + 2, jnp.int32)
                c3 = jnp.full((L,), d + 3, jnp.int32)
                a0 = a0 + plsc.load_gather(ra, [rowi, c0]) * plsc.load_gather(rb, [rowi, c0])
                a1 = a1 + plsc.load_gather(ra, [rowi, c1]) * plsc.load_gather(rb, [rowi, c1])
                a2 = a2 + plsc.load_gather(ra, [rowi, c2]) * plsc.load_gather(rb, [rowi, c2])
                a3 = a3 + plsc.load_gather(ra, [rowi, c3]) * plsc.load_gather(rb, [rowi, c3])
            lbuf[pl.ds(j * CB + g * L, L)] = (a0 + a1) + (a2 + a3)
            return gcarry

        lax.fori_loop(0, CB // L, group, 0)

    gather(0, ra0, rb0, sem0)

    def body(j, carry):
        gwait(j, ra0, rb0, sem0)
        gather(j + 1, ra1, rb1, sem1)
        compute(j, ra0, rb0)
        gwait(j + 1, ra1, rb1, sem1)

        @pl.when(j + 2 < KB2)
        def _():
            gather(j + 2, ra0, rb0, sem0)

        compute(j + 1, ra1, rb1)
        return carry

    lax.fori_loop(0, KB2 // 2, lambda t, cc: body(t * 2, cc), 0)
    pltpu.sync_copy(lbuf, out_hbm.at[wid])


def _dinv_block(degp_blk, block_i):
    deg = jnp.sum(degp_blk, axis=0)[:, None] + 1.0
    rows = lax.broadcasted_iota(jnp.int32, (BN, 1), 0) + block_i * BN
    return jnp.where(rows < N, lax.rsqrt(deg), 0.0)


def _tc1_body(x_ref, w_ref, degp_ref, h1p_ref):
    i = pl.program_id(0)
    dinv = _dinv_block(degp_ref[...], i)
    h = jnp.dot(x_ref[...], w_ref[...], preferred_element_type=jnp.float32)
    h1p_ref[...] = h * dinv


def _tc2_body(p_ref, h1p_ref, degp_ref, w2_ref, b1_ref, h2p_ref):
    i = pl.program_id(0)
    dinv = _dinv_block(degp_ref[...], i)
    agg = p_ref[0] + p_ref[1] + h1p_ref[...]
    z1 = jnp.maximum(dinv * agg + b1_ref[...], 0.0)
    h2p_ref[...] = jnp.dot(z1, w2_ref[...], preferred_element_type=jnp.float32) * dinv


def _tc3_body(q_ref, h2p_ref, degp_ref, b2_ref, lat_ref):
    i = pl.program_id(0)
    dinv = _dinv_block(degp_ref[...], i)
    lat_ref[...] = dinv * (q_ref[0] + q_ref[1] + h2p_ref[...]) + b2_ref[...]


def _tc1(x_p, W1, degp):
    return pl.pallas_call(
        _tc1_body,
        grid=(NP // BN,),
        in_specs=[
            pl.BlockSpec((BN, DF), lambda i: (i, 0)),
            pl.BlockSpec((DF, H), lambda i: (0, 0)),
            pl.BlockSpec((NW, BN), lambda i: (0, i)),
        ],
        out_specs=pl.BlockSpec((BN, H), lambda i: (i, 0)),
        out_shape=jax.ShapeDtypeStruct((NP, H), jnp.float32),
    )(x_p, W1, degp)


def _tc2(p, h1p, degp, W2, b1):
    return pl.pallas_call(
        _tc2_body,
        grid=(NP // BN,),
        in_specs=[
            pl.BlockSpec((NC, BN, H), lambda i: (0, i, 0)),
            pl.BlockSpec((BN, H), lambda i: (i, 0)),
            pl.BlockSpec((NW, BN), lambda i: (0, i)),
            pl.BlockSpec((H, OUT), lambda i: (0, 0)),
            pl.BlockSpec((1, H), lambda i: (0, 0)),
        ],
        out_specs=pl.BlockSpec((BN, OUT), lambda i: (i, 0)),
        out_shape=jax.ShapeDtypeStruct((NP, OUT), jnp.float32),
    )(p, h1p, degp, W2, b1)


def _tc3(q, h2p, degp, b2):
    return pl.pallas_call(
        _tc3_body,
        grid=(NP // BN,),
        in_specs=[
            pl.BlockSpec((NC, BN, OUT), lambda i: (0, i, 0)),
            pl.BlockSpec((BN, OUT), lambda i: (i, 0)),
            pl.BlockSpec((NW, BN), lambda i: (0, i)),
            pl.BlockSpec((1, OUT), lambda i: (0, 0)),
        ],
        out_specs=pl.BlockSpec((BN, OUT), lambda i: (i, 0)),
        out_shape=jax.ShapeDtypeStruct((NP, OUT), jnp.float32),
    )(q, h2p, degp, b2)


def _pad_idx(idx, kch, cb):
    tot = NW * kch * cb
    p = jnp.full((tot,), PAD_NODE, jnp.int32).at[: idx.shape[0]].set(idx)
    return p.reshape(NW, kch, cb)


def kernel(x, edge_index, neg_edge, W1, b1, W2, b2):
    src1 = _pad_idx(edge_index[0], KCH1, CB1)
    dst1 = _pad_idx(edge_index[1], KCH1, CB1)
    src2 = _pad_idx(edge_index[0], KCH, CB)
    dst2 = _pad_idx(edge_index[1], KCH, CB)

    x_p = jnp.zeros((NP, DF), jnp.float32).at[:N].set(x)
    zeros_h = jnp.zeros((STRIPE, H), jnp.float32)
    zeros_o = jnp.zeros((STRIPE, OUT), jnp.float32)

    degp = _deg_kernel(dst2)
    h1p = _tc1(x_p, W1, degp)
    p1 = _agg_h(h1p, src1, dst1, zeros_h)
    h2p = _tc2(p1, h1p, degp, W2, b1.reshape(1, H))
    p2 = _agg_out(h2p, src2, dst2, zeros_o)
    latent = _tc3(p2, h2p, degp, b2.reshape(1, OUT))

    ia = _pad_idx(jnp.concatenate([edge_index[0], neg_edge[0]]), KB2, CB)
    ib = _pad_idx(jnp.concatenate([edge_index[1], neg_edge[1]]), KB2, CB)
    logits = _dec_kernel(latent, ia, ib)
    return logits.reshape(-1)[:E2]


# speedup vs baseline: 9.8179x; 1.4398x over previous
import functools

import jax
import jax.numpy as jnp
from jax import lax
from jax.experimental import pallas as pl
from jax.experimental.pallas import tpu as pltpu
from jax.experimental.pallas import tpu_sc as plsc

N = 10000
DF = 128
H = 128
OUT = 64
E = 320000
E2 = 2 * E

NC = 2
NS = 16
NW = NC * NS
L = 16

NP = 10240
STRIPE = NP // NS
PAD_NODE = NP - 1

CB = 128
KCH = 80
CB1 = 64
KCH1 = 160
KB2 = 158

BN = 1024

_mesh = plsc.VectorSubcoreMesh(core_axis_name="c", subcore_axis_name="s")
_sc_params = pltpu.CompilerParams(
    needs_layout_passes=False, use_tc_tiling_on_sc=False
)


@functools.partial(
    pl.kernel,
    out_type=jax.ShapeDtypeStruct((NW, NP), jnp.float32),
    mesh=_mesh,
    scratch_types=[
        pltpu.VMEM((KCH, CB), jnp.int32),
        pltpu.VMEM((NP,), jnp.float32),
    ],
    compiler_params=_sc_params,
)
def _deg_kernel(dst_hbm, degp_hbm, idx_v, deg_v):
    c = lax.axis_index("c")
    s = lax.axis_index("s")
    wid = s * NC + c
    pltpu.sync_copy(dst_hbm.at[wid], idx_v)
    zero = jnp.zeros((L,), jnp.float32)

    def zbody(i, carry):
        deg_v[pl.ds(i * L, L)] = zero
        return carry

    lax.fori_loop(0, NP // L, zbody, 0)
    ones = jnp.ones((L,), jnp.float32)

    def cbody(j, carry):
        for g in range(CB // L):
            idx16 = idx_v[j, pl.ds(g * L, L)]
            plsc.addupdate_scatter(deg_v, [idx16], ones)
        return carry

    lax.fori_loop(0, KCH, cbody, 0)
    pltpu.sync_copy(deg_v, degp_hbm.at[wid])


def _make_agg(dim, kch, cb):
    @functools.partial(
        pl.kernel,
        out_type=jax.ShapeDtypeStruct((NC, NP, dim), jnp.float32),
        mesh=_mesh,
        scratch_types=[
            pltpu.VMEM((kch, cb), jnp.int32),
            pltpu.VMEM((kch, cb), jnp.int32),
            pltpu.VMEM((cb, dim), jnp.float32),
            pltpu.VMEM((cb, dim), jnp.float32),
            pltpu.VMEM_SHARED((NP, dim), jnp.float32),
            pltpu.SemaphoreType.DMA,
            pltpu.SemaphoreType.DMA,
        ],
        name=f"edge_agg_{dim}",
        compiler_params=_sc_params,
    )
    def agg(table_hbm, src_hbm, dst_hbm, zeros_hbm, out_hbm, src_v, dst_v,
            rows0, rows1, shared, sem0, sem1):
        c = lax.axis_index("c")
        s = lax.axis_index("s")
        wid = s * NC + c
        stripe = pl.ds(s * STRIPE, STRIPE)
        pltpu.sync_copy(zeros_hbm, shared.at[stripe])
        pltpu.sync_copy(src_hbm.at[wid], src_v)
        pltpu.sync_copy(dst_hbm.at[wid], dst_v)
        plsc.subcore_barrier()

        def gather(j, rows, sem):
            pltpu.async_copy(table_hbm.at[src_v.at[j]], rows, sem)

        def gwait(j, rows, sem):
            pltpu.make_async_copy(table_hbm.at[src_v.at[j]], rows, sem).wait()

        gather(0, rows0, sem0)

        def body(j, carry):
            gwait(j, rows0, sem0)
            gather(j + 1, rows1, sem1)
            pltpu.sync_copy(rows0, shared.at[dst_v.at[j]], add=True)
            gwait(j + 1, rows1, sem1)

            @pl.when(j + 2 < kch)
            def _():
                gather(j + 2, rows0, sem0)

            pltpu.sync_copy(rows1, shared.at[dst_v.at[j + 1]], add=True)
            return carry

        lax.fori_loop(0, kch // 2, lambda t, cc: body(t * 2, cc), 0)
        plsc.subcore_barrier()
        pltpu.sync_copy(shared.at[stripe], out_hbm.at[c, stripe])

    return agg


_agg_h = _make_agg(H, KCH1, CB1)
_agg_out = _make_agg(OUT, KCH, CB)


@functools.partial(
    pl.kernel,
    out_type=jax.ShapeDtypeStruct((NW, KB2 * CB), jnp.float32),
    mesh=_mesh,
    scratch_types=[
        pltpu.VMEM((KB2, CB), jnp.int32),
        pltpu.VMEM((KB2, CB), jnp.int32),
        pltpu.VMEM((CB, OUT), jnp.float32),
        pltpu.VMEM((CB, OUT), jnp.float32),
        pltpu.VMEM((CB, OUT), jnp.float32),
        pltpu.VMEM((CB, OUT), jnp.float32),
        pltpu.VMEM((KB2 * CB,), jnp.float32),
        pltpu.SemaphoreType.DMA,
        pltpu.SemaphoreType.DMA,
    ],
    name="edge_decoder",
    compiler_params=_sc_params,
)
def _dec_kernel(latent_hbm, ia_hbm, ib_hbm, out_hbm, ia_v, ib_v,
                ra0, rb0, ra1, rb1, lbuf, sem0, sem1):
    c = lax.axis_index("c")
    s = lax.axis_index("s")
    wid = s * NC + c
    pltpu.sync_copy(ia_hbm.at[wid], ia_v)
    pltpu.sync_copy(ib_hbm.at[wid], ib_v)
    iota = lax.iota(jnp.int32, L)

    def gather(j, ra, rb, sem):
        pltpu.async_copy(latent_hbm.at[ia_v.at[j]], ra, sem)
        pltpu.async_copy(latent_hbm.at[ib_v.at[j]], rb, sem)

    def gwait(j, ra, rb, sem):
        pltpu.make_async_copy(latent_hbm.at[ia_v.at[j]], ra, sem).wait()
        pltpu.make_async_copy(latent_hbm.at[ib_v.at[j]], rb, sem).wait()

    def compute(j, ra, rb):
        def group(g, gcarry):
            rowi = g * L + iota
            a0 = jnp.zeros((L,), jnp.float32)
            a1 = a0
            a2 = a0
            a3 = a0
            for d in range(0, OUT, 4):
                c0 = (iota + d) & (OUT - 1)
                c1 = (iota + (d + 1)) & (OUT - 1)
                c2 = (iota + (d + 2)) & (OUT - 1)
                c3 = (iota + (d + 3)) & (OUT - 1)
                a0 = a0 + plsc.load_gather(ra, [rowi, c0]) * plsc.load_gather(rb, [rowi, c0])
                a1 = a1 + plsc.load_gather(ra, [rowi, c1]) * plsc.load_gather(rb, [rowi, c1])
                a2 = a2 + plsc.load_gather(ra, [rowi, c2]) * plsc.load_gather(rb, [rowi, c2])
                a3 = a3 + plsc.load_gather(ra, [rowi, c3]) * plsc.load_gather(rb, [rowi, c3])
            lbuf[pl.ds(j * CB + g * L, L)] = (a0 + a1) + (a2 + a3)
            return gcarry

        lax.fori_loop(0, CB // L, group, 0)

    gather(0, ra0, rb0, sem0)

    def body(j, carry):
        gwait(j, ra0, rb0, sem0)
        gather(j + 1, ra1, rb1, sem1)
        compute(j, ra0, rb0)
        gwait(j + 1, ra1, rb1, sem1)

        @pl.when(j + 2 < KB2)
        def _():
            gather(j + 2, ra0, rb0, sem0)

        compute(j + 1, ra1, rb1)
        return carry

    lax.fori_loop(0, KB2 // 2, lambda t, cc: body(t * 2, cc), 0)
    pltpu.sync_copy(lbuf, out_hbm.at[wid])


def _dinv_block(degp_blk, block_i):
    deg = jnp.sum(degp_blk, axis=0)[:, None] + 1.0
    rows = lax.broadcasted_iota(jnp.int32, (BN, 1), 0) + block_i * BN
    return jnp.where(rows < N, lax.rsqrt(deg), 0.0)


def _tc1_body(x_ref, w_ref, degp_ref, h1p_ref):
    i = pl.program_id(0)
    dinv = _dinv_block(degp_ref[...], i)
    h = jnp.dot(x_ref[...], w_ref[...], preferred_element_type=jnp.float32)
    h1p_ref[...] = h * dinv


def _tc2_body(p_ref, h1p_ref, degp_ref, w2_ref, b1_ref, h2p_ref):
    i = pl.program_id(0)
    dinv = _dinv_block(degp_ref[...], i)
    agg = p_ref[0] + p_ref[1] + h1p_ref[...]
    z1 = jnp.maximum(dinv * agg + b1_ref[...], 0.0)
    h2p_ref[...] = jnp.dot(z1, w2_ref[...], preferred_element_type=jnp.float32) * dinv


def _tc3_body(q_ref, h2p_ref, degp_ref, b2_ref, lat_ref):
    i = pl.program_id(0)
    dinv = _dinv_block(degp_ref[...], i)
    lat_ref[...] = dinv * (q_ref[0] + q_ref[1] + h2p_ref[...]) + b2_ref[...]


def _tc1(x_p, W1, degp):
    return pl.pallas_call(
        _tc1_body,
        grid=(NP // BN,),
        in_specs=[
            pl.BlockSpec((BN, DF), lambda i: (i, 0)),
            pl.BlockSpec((DF, H), lambda i: (0, 0)),
            pl.BlockSpec((NW, BN), lambda i: (0, i)),
        ],
        out_specs=pl.BlockSpec((BN, H), lambda i: (i, 0)),
        out_shape=jax.ShapeDtypeStruct((NP, H), jnp.float32),
    )(x_p, W1, degp)


def _tc2(p, h1p, degp, W2, b1):
    return pl.pallas_call(
        _tc2_body,
        grid=(NP // BN,),
        in_specs=[
            pl.BlockSpec((NC, BN, H), lambda i: (0, i, 0)),
            pl.BlockSpec((BN, H), lambda i: (i, 0)),
            pl.BlockSpec((NW, BN), lambda i: (0, i)),
            pl.BlockSpec((H, OUT), lambda i: (0, 0)),
            pl.BlockSpec((1, H), lambda i: (0, 0)),
        ],
        out_specs=pl.BlockSpec((BN, OUT), lambda i: (i, 0)),
        out_shape=jax.ShapeDtypeStruct((NP, OUT), jnp.float32),
    )(p, h1p, degp, W2, b1)


def _tc3(q, h2p, degp, b2):
    return pl.pallas_call(
        _tc3_body,
        grid=(NP // BN,),
        in_specs=[
            pl.BlockSpec((NC, BN, OUT), lambda i: (0, i, 0)),
            pl.BlockSpec((BN, OUT), lambda i: (i, 0)),
            pl.BlockSpec((NW, BN), lambda i: (0, i)),
            pl.BlockSpec((1, OUT), lambda i: (0, 0)),
        ],
        out_specs=pl.BlockSpec((BN, OUT), lambda i: (i, 0)),
        out_shape=jax.ShapeDtypeStruct((NP, OUT), jnp.float32),
    )(q, h2p, degp, b2)


def _pad_idx(idx, kch, cb):
    tot = NW * kch * cb
    p = jnp.full((tot,), PAD_NODE, jnp.int32).at[: idx.shape[0]].set(idx)
    return p.reshape(NW, kch, cb)


def kernel(x, edge_index, neg_edge, W1, b1, W2, b2):
    src1 = _pad_idx(edge_index[0], KCH1, CB1)
    dst1 = _pad_idx(edge_index[1], KCH1, CB1)
    src2 = _pad_idx(edge_index[0], KCH, CB)
    dst2 = _pad_idx(edge_index[1], KCH, CB)

    x_p = jnp.zeros((NP, DF), jnp.float32).at[:N].set(x)
    zeros_h = jnp.zeros((STRIPE, H), jnp.float32)
    zeros_o = jnp.zeros((STRIPE, OUT), jnp.float32)

    degp = _deg_kernel(dst2)
    h1p = _tc1(x_p, W1, degp)
    p1 = _agg_h(h1p, src1, dst1, zeros_h)
    h2p = _tc2(p1, h1p, degp, W2, b1.reshape(1, H))
    p2 = _agg_out(h2p, src2, dst2, zeros_o)
    latent = _tc3(p2, h2p, degp, b2.reshape(1, OUT))

    ia = _pad_idx(jnp.concatenate([edge_index[0], neg_edge[0]]), KB2, CB)
    ib = _pad_idx(jnp.concatenate([edge_index[1], neg_edge[1]]), KB2, CB)
    logits = _dec_kernel(latent, ia, ib)
    return logits.reshape(-1)[:E2]


# speedup vs baseline: 11.0163x; 1.1221x over previous
import functools

import jax
import jax.numpy as jnp
from jax import lax
from jax.experimental import pallas as pl
from jax.experimental.pallas import tpu as pltpu
from jax.experimental.pallas import tpu_sc as plsc

N = 10000
DF = 128
H = 128
OUT = 64
E = 320000
E2 = 2 * E

NC = 2
NS = 16
NW = NC * NS
L = 16

NP = 10240
STRIPE = NP // NS
PAD_NODE = NP - 1

CB = 128
KCH = 80
CB1 = 64
KCH1 = 160
KB2 = 158

BN = 1024

_mesh = plsc.VectorSubcoreMesh(core_axis_name="c", subcore_axis_name="s")
_sc_params = pltpu.CompilerParams(
    needs_layout_passes=False, use_tc_tiling_on_sc=False
)


@functools.partial(
    pl.kernel,
    out_type=jax.ShapeDtypeStruct((NW, NP), jnp.float32),
    mesh=_mesh,
    scratch_types=[
        pltpu.VMEM((KCH, CB), jnp.int32),
        pltpu.VMEM((NP,), jnp.float32),
    ],
    compiler_params=_sc_params,
)
def _deg_kernel(dst_hbm, degp_hbm, idx_v, deg_v):
    c = lax.axis_index("c")
    s = lax.axis_index("s")
    wid = s * NC + c
    pltpu.sync_copy(dst_hbm.at[wid], idx_v)
    zero = jnp.zeros((L,), jnp.float32)

    def zbody(i, carry):
        deg_v[pl.ds(i * L, L)] = zero
        return carry

    lax.fori_loop(0, NP // L, zbody, 0)
    ones = jnp.ones((L,), jnp.float32)

    def cbody(j, carry):
        for g in range(CB // L):
            idx16 = idx_v[j, pl.ds(g * L, L)]
            plsc.addupdate_scatter(deg_v, [idx16], ones)
        return carry

    lax.fori_loop(0, KCH, cbody, 0)
    pltpu.sync_copy(deg_v, degp_hbm.at[wid])


def _make_agg(dim, kch, cb, table_in_spmem):
    scratch = [
        pltpu.VMEM((kch, cb), jnp.int32),
        pltpu.VMEM((kch, cb), jnp.int32),
        pltpu.VMEM((cb, dim), jnp.float32),
        pltpu.VMEM((cb, dim), jnp.float32),
        pltpu.VMEM_SHARED((NP, dim), jnp.float32),
        pltpu.SemaphoreType.DMA,
        pltpu.SemaphoreType.DMA,
    ]
    if table_in_spmem:
        scratch.append(pltpu.VMEM_SHARED((NP, dim), jnp.float32))

    @functools.partial(
        pl.kernel,
        out_type=jax.ShapeDtypeStruct((NC, NP, dim), jnp.float32),
        mesh=_mesh,
        scratch_types=scratch,
        name=f"edge_agg_{dim}",
        compiler_params=_sc_params,
    )
    def agg(table_hbm, src_hbm, dst_hbm, zeros_hbm, out_hbm, src_v, dst_v,
            rows0, rows1, shared, sem0, sem1, *rest):
        c = lax.axis_index("c")
        s = lax.axis_index("s")
        wid = s * NC + c
        stripe = pl.ds(s * STRIPE, STRIPE)
        pltpu.sync_copy(zeros_hbm, shared.at[stripe])
        pltpu.sync_copy(src_hbm.at[wid], src_v)
        pltpu.sync_copy(dst_hbm.at[wid], dst_v)
        if table_in_spmem:
            table = rest[0]
            pltpu.sync_copy(table_hbm.at[stripe], table.at[stripe])
        else:
            table = table_hbm
        plsc.subcore_barrier()

        def gather(j, rows, sem):
            pltpu.async_copy(table.at[src_v.at[j]], rows, sem)

        def gwait(j, rows, sem):
            pltpu.make_async_copy(table.at[src_v.at[j]], rows, sem).wait()

        gather(0, rows0, sem0)

        def body(j, carry):
            gwait(j, rows0, sem0)
            gather(j + 1, rows1, sem1)
            pltpu.sync_copy(rows0, shared.at[dst_v.at[j]], add=True)
            gwait(j + 1, rows1, sem1)

            @pl.when(j + 2 < kch)
            def _():
                gather(j + 2, rows0, sem0)

            pltpu.sync_copy(rows1, shared.at[dst_v.at[j + 1]], add=True)
            return carry

        lax.fori_loop(0, kch // 2, lambda t, cc: body(t * 2, cc), 0)
        plsc.subcore_barrier()
        pltpu.sync_copy(shared.at[stripe], out_hbm.at[c, stripe])

    return agg


_agg_h = _make_agg(H, KCH1, CB1, False)
_agg_out = _make_agg(OUT, KCH, CB, True)


@functools.partial(
    pl.kernel,
    out_type=jax.ShapeDtypeStruct((NW, KB2 * CB), jnp.float32),
    mesh=_mesh,
    scratch_types=[
        pltpu.VMEM((KB2, CB), jnp.int32),
        pltpu.VMEM((KB2, CB), jnp.int32),
        pltpu.VMEM((CB, OUT), jnp.float32),
        pltpu.VMEM((CB, OUT), jnp.float32),
        pltpu.VMEM((CB, OUT), jnp.float32),
        pltpu.VMEM((CB, OUT), jnp.float32),
        pltpu.VMEM((KB2 * CB,), jnp.float32),
        pltpu.SemaphoreType.DMA,
        pltpu.SemaphoreType.DMA,
    ],
    name="edge_decoder",
    compiler_params=_sc_params,
)
def _dec_kernel(latent_hbm, ia_hbm, ib_hbm, out_hbm, ia_v, ib_v,
                ra0, rb0, ra1, rb1, lbuf, sem0, sem1):
    c = lax.axis_index("c")
    s = lax.axis_index("s")
    wid = s * NC + c
    pltpu.sync_copy(ia_hbm.at[wid], ia_v)
    pltpu.sync_copy(ib_hbm.at[wid], ib_v)
    iota = lax.iota(jnp.int32, L)

    def gather(j, ra, rb, sem):
        pltpu.async_copy(latent_hbm.at[ia_v.at[j]], ra, sem)
        pltpu.async_copy(latent_hbm.at[ib_v.at[j]], rb, sem)

    def gwait(j, ra, rb, sem):
        pltpu.make_async_copy(latent_hbm.at[ia_v.at[j]], ra, sem).wait()
        pltpu.make_async_copy(latent_hbm.at[ib_v.at[j]], rb, sem).wait()

    def compute(j, ra, rb):
        def group(g, gcarry):
            rowi = g * L + iota
            a0 = jnp.zeros((L,), jnp.float32)
            a1 = a0
            a2 = a0
            a3 = a0
            for d in range(0, OUT, 4):
                c0 = (iota + d) & (OUT - 1)
                c1 = (iota + (d + 1)) & (OUT - 1)
                c2 = (iota + (d + 2)) & (OUT - 1)
                c3 = (iota + (d + 3)) & (OUT - 1)
                a0 = a0 + plsc.load_gather(ra, [rowi, c0]) * plsc.load_gather(rb, [rowi, c0])
                a1 = a1 + plsc.load_gather(ra, [rowi, c1]) * plsc.load_gather(rb, [rowi, c1])
                a2 = a2 + plsc.load_gather(ra, [rowi, c2]) * plsc.load_gather(rb, [rowi, c2])
                a3 = a3 + plsc.load_gather(ra, [rowi, c3]) * plsc.load_gather(rb, [rowi, c3])
            lbuf[pl.ds(j * CB + g * L, L)] = (a0 + a1) + (a2 + a3)
            return gcarry

        lax.fori_loop(0, CB // L, group, 0)

    gather(0, ra0, rb0, sem0)

    def body(j, carry):
        gwait(j, ra0, rb0, sem0)
        gather(j + 1, ra1, rb1, sem1)
        compute(j, ra0, rb0)
        gwait(j + 1, ra1, rb1, sem1)

        @pl.when(j + 2 < KB2)
        def _():
            gather(j + 2, ra0, rb0, sem0)

        compute(j + 1, ra1, rb1)
        return carry

    lax.fori_loop(0, KB2 // 2, lambda t, cc: body(t * 2, cc), 0)
    pltpu.sync_copy(lbuf, out_hbm.at[wid])


def _dinv_block(degp_blk, block_i):
    deg = jnp.sum(degp_blk, axis=0)[:, None] + 1.0
    rows = lax.broadcasted_iota(jnp.int32, (BN, 1), 0) + block_i * BN
    return jnp.where(rows < N, lax.rsqrt(deg), 0.0)


def _tc1_body(x_ref, w_ref, degp_ref, h1p_ref):
    i = pl.program_id(0)
    dinv = _dinv_block(degp_ref[...], i)
    h = jnp.dot(x_ref[...], w_ref[...], preferred_element_type=jnp.float32)
    h1p_ref[...] = h * dinv


def _tc2_body(p_ref, h1p_ref, degp_ref, w2_ref, b1_ref, h2p_ref):
    i = pl.program_id(0)
    dinv = _dinv_block(degp_ref[...], i)
    agg = p_ref[0] + p_ref[1] + h1p_ref[...]
    z1 = jnp.maximum(dinv * agg + b1_ref[...], 0.0)
    h2p_ref[...] = jnp.dot(z1, w2_ref[...], preferred_element_type=jnp.float32) * dinv


def _tc3_body(q_ref, h2p_ref, degp_ref, b2_ref, lat_ref):
    i = pl.program_id(0)
    dinv = _dinv_block(degp_ref[...], i)
    lat_ref[...] = dinv * (q_ref[0] + q_ref[1] + h2p_ref[...]) + b2_ref[...]


def _tc1(x_p, W1, degp):
    return pl.pallas_call(
        _tc1_body,
        grid=(NP // BN,),
        in_specs=[
            pl.BlockSpec((BN, DF), lambda i: (i, 0)),
            pl.BlockSpec((DF, H), lambda i: (0, 0)),
            pl.BlockSpec((NW, BN), lambda i: (0, i)),
        ],
        out_specs=pl.BlockSpec((BN, H), lambda i: (i, 0)),
        out_shape=jax.ShapeDtypeStruct((NP, H), jnp.float32),
    )(x_p, W1, degp)


def _tc2(p, h1p, degp, W2, b1):
    return pl.pallas_call(
        _tc2_body,
        grid=(NP // BN,),
        in_specs=[
            pl.BlockSpec((NC, BN, H), lambda i: (0, i, 0)),
            pl.BlockSpec((BN, H), lambda i: (i, 0)),
            pl.BlockSpec((NW, BN), lambda i: (0, i)),
            pl.BlockSpec((H, OUT), lambda i: (0, 0)),
            pl.BlockSpec((1, H), lambda i: (0, 0)),
        ],
        out_specs=pl.BlockSpec((BN, OUT), lambda i: (i, 0)),
        out_shape=jax.ShapeDtypeStruct((NP, OUT), jnp.float32),
    )(p, h1p, degp, W2, b1)


def _tc3(q, h2p, degp, b2):
    return pl.pallas_call(
        _tc3_body,
        grid=(NP // BN,),
        in_specs=[
            pl.BlockSpec((NC, BN, OUT), lambda i: (0, i, 0)),
            pl.BlockSpec((BN, OUT), lambda i: (i, 0)),
            pl.BlockSpec((NW, BN), lambda i: (0, i)),
            pl.BlockSpec((1, OUT), lambda i: (0, 0)),
        ],
        out_specs=pl.BlockSpec((BN, OUT), lambda i: (i, 0)),
        out_shape=jax.ShapeDtypeStruct((NP, OUT), jnp.float32),
    )(q, h2p, degp, b2)


def _pad_idx(idx, kch, cb):
    tot = NW * kch * cb
    p = jnp.full((tot,), PAD_NODE, jnp.int32).at[: idx.shape[0]].set(idx)
    return p.reshape(NW, kch, cb)


def kernel(x, edge_index, neg_edge, W1, b1, W2, b2):
    src1 = _pad_idx(edge_index[0], KCH1, CB1)
    dst1 = _pad_idx(edge_index[1], KCH1, CB1)
    src2 = _pad_idx(edge_index[0], KCH, CB)
    dst2 = _pad_idx(edge_index[1], KCH, CB)

    x_p = jnp.zeros((NP, DF), jnp.float32).at[:N].set(x)
    zeros_h = jnp.zeros((STRIPE, H), jnp.float32)
    zeros_o = jnp.zeros((STRIPE, OUT), jnp.float32)

    degp = _deg_kernel(dst2)
    h1p = _tc1(x_p, W1, degp)
    p1 = _agg_h(h1p, src1, dst1, zeros_h)
    h2p = _tc2(p1, h1p, degp, W2, b1.reshape(1, H))
    p2 = _agg_out(h2p, src2, dst2, zeros_o)
    latent = _tc3(p2, h2p, degp, b2.reshape(1, OUT))

    ia = _pad_idx(jnp.concatenate([edge_index[0], neg_edge[0]]), KB2, CB)
    ib = _pad_idx(jnp.concatenate([edge_index[1], neg_edge[1]]), KB2, CB)
    logits = _dec_kernel(latent, ia, ib)
    return logits.reshape(-1)[:E2]


# speedup vs baseline: 14.7607x; 1.3399x over previous
import functools

import jax
import jax.numpy as jnp
from jax import lax
from jax.experimental import pallas as pl
from jax.experimental.pallas import tpu as pltpu
from jax.experimental.pallas import tpu_sc as plsc

N = 10000
DF = 128
H = 128
OUT = 64
E = 320000
E2 = 2 * E

NC = 2
NS = 16
NW = NC * NS
L = 16

NP = 10240
STRIPE = NP // NS
PAD_NODE = NP - 1

CB = 128
KCH = 80
CB1 = 64
KCH1 = 160
KB2 = 158

BN = 1024

_mesh = plsc.VectorSubcoreMesh(core_axis_name="c", subcore_axis_name="s")
_sc_params = pltpu.CompilerParams(
    needs_layout_passes=False, use_tc_tiling_on_sc=False
)


@functools.partial(
    pl.kernel,
    out_type=jax.ShapeDtypeStruct((NW, NP), jnp.float32),
    mesh=_mesh,
    scratch_types=[
        pltpu.VMEM((KCH, CB), jnp.int32),
        pltpu.VMEM((NP,), jnp.float32),
    ],
    compiler_params=_sc_params,
)
def _deg_kernel(dst_hbm, degp_hbm, idx_v, deg_v):
    c = lax.axis_index("c")
    s = lax.axis_index("s")
    wid = s * NC + c
    pltpu.sync_copy(dst_hbm.at[wid], idx_v)
    zero = jnp.zeros((L,), jnp.float32)

    def zbody(i, carry):
        deg_v[pl.ds(i * L, L)] = zero
        return carry

    lax.fori_loop(0, NP // L, zbody, 0)
    ones = jnp.ones((L,), jnp.float32)

    def cbody(j, carry):
        for g in range(CB // L):
            idx16 = idx_v[j, pl.ds(g * L, L)]
            plsc.addupdate_scatter(deg_v, [idx16], ones)
        return carry

    lax.fori_loop(0, KCH, cbody, 0)
    pltpu.sync_copy(deg_v, degp_hbm.at[wid])


def _make_agg(dim, kch, cb, table_in_spmem):
    scratch = [
        pltpu.VMEM((kch, cb), jnp.int32),
        pltpu.VMEM((kch, cb), jnp.int32),
        pltpu.VMEM((cb, dim), jnp.float32),
        pltpu.VMEM((cb, dim), jnp.float32),
        pltpu.VMEM_SHARED((NP, dim), jnp.float32),
        pltpu.SemaphoreType.DMA,
        pltpu.SemaphoreType.DMA,
    ]
    if table_in_spmem:
        scratch.append(pltpu.VMEM_SHARED((NP, dim), jnp.float32))

    @functools.partial(
        pl.kernel,
        out_type=jax.ShapeDtypeStruct((NC, NP, dim), jnp.float32),
        mesh=_mesh,
        scratch_types=scratch,
        name=f"edge_agg_{dim}",
        compiler_params=_sc_params,
    )
    def agg(table_hbm, src_hbm, dst_hbm, zeros_hbm, out_hbm, src_v, dst_v,
            rows0, rows1, shared, sem0, sem1, *rest):
        c = lax.axis_index("c")
        s = lax.axis_index("s")
        wid = s * NC + c
        stripe = pl.ds(s * STRIPE, STRIPE)
        pltpu.sync_copy(zeros_hbm, shared.at[stripe])
        pltpu.sync_copy(src_hbm.at[wid], src_v)
        pltpu.sync_copy(dst_hbm.at[wid], dst_v)
        if table_in_spmem:
            table = rest[0]
            pltpu.sync_copy(table_hbm.at[stripe], table.at[stripe])
        else:
            table = table_hbm
        plsc.subcore_barrier()

        def gather(j, rows, sem):
            pltpu.async_copy(table.at[src_v.at[j]], rows, sem)

        def gwait(j, rows, sem):
            pltpu.make_async_copy(table.at[src_v.at[j]], rows, sem).wait()

        gather(0, rows0, sem0)

        def body(j, carry):
            gwait(j, rows0, sem0)
            gather(j + 1, rows1, sem1)
            pltpu.sync_copy(rows0, shared.at[dst_v.at[j]], add=True)
            gwait(j + 1, rows1, sem1)

            @pl.when(j + 2 < kch)
            def _():
                gather(j + 2, rows0, sem0)

            pltpu.sync_copy(rows1, shared.at[dst_v.at[j + 1]], add=True)
            return carry

        lax.fori_loop(0, kch // 2, lambda t, cc: body(t * 2, cc), 0)
        plsc.subcore_barrier()
        pltpu.sync_copy(shared.at[stripe], out_hbm.at[c, stripe])

    return agg


_agg_h = _make_agg(H, KCH1, CB1, False)
_agg_out = _make_agg(OUT, KCH, CB, True)


@functools.partial(
    pl.kernel,
    out_type=jax.ShapeDtypeStruct((NW, KB2 * CB), jnp.float32),
    mesh=_mesh,
    scratch_types=[
        pltpu.VMEM((KB2, CB), jnp.int32),
        pltpu.VMEM((KB2, CB), jnp.int32),
        pltpu.VMEM((CB, OUT), jnp.float32),
        pltpu.VMEM((CB, OUT), jnp.float32),
        pltpu.VMEM((CB, OUT), jnp.float32),
        pltpu.VMEM((CB, OUT), jnp.float32),
        pltpu.VMEM((CB,), jnp.float32),
        pltpu.VMEM((CB,), jnp.float32),
        pltpu.VMEM_SHARED((NP, OUT), jnp.float32),
        pltpu.SemaphoreType.DMA,
        pltpu.SemaphoreType.DMA,
        pltpu.SemaphoreType.DMA,
        pltpu.SemaphoreType.DMA,
    ],
    name="edge_decoder",
    compiler_params=_sc_params,
)
def _dec_kernel(latent_hbm, ia_hbm, ib_hbm, out_hbm, ia_v, ib_v,
                ra0, rb0, ra1, rb1, ob0, ob1, table,
                sem0, sem1, osem0, osem1):
    c = lax.axis_index("c")
    s = lax.axis_index("s")
    wid = s * NC + c
    stripe = pl.ds(s * STRIPE, STRIPE)
    pltpu.sync_copy(latent_hbm.at[stripe], table.at[stripe])
    pltpu.sync_copy(ia_hbm.at[wid], ia_v)
    pltpu.sync_copy(ib_hbm.at[wid], ib_v)
    plsc.subcore_barrier()
    iota = lax.iota(jnp.int32, L)

    def gather(j, ra, rb, sem):
        pltpu.async_copy(table.at[ia_v.at[j]], ra, sem)
        pltpu.async_copy(table.at[ib_v.at[j]], rb, sem)

    def gwait(j, ra, rb, sem):
        pltpu.make_async_copy(table.at[ia_v.at[j]], ra, sem).wait()
        pltpu.make_async_copy(table.at[ib_v.at[j]], rb, sem).wait()

    def owrite(j, ob, osem):
        pltpu.async_copy(ob, out_hbm.at[wid, pl.ds(j * CB, CB)], osem)

    def owait(j, ob, osem):
        pltpu.make_async_copy(
            ob, out_hbm.at[wid, pl.ds(j * CB, CB)], osem
        ).wait()

    def compute(j, ra, rb, ob):
        def group(g, gcarry):
            rowi = g * L + iota
            a0 = jnp.zeros((L,), jnp.float32)
            a1 = a0
            a2 = a0
            a3 = a0
            for d in range(0, OUT, 4):
                c0 = (iota + d) & (OUT - 1)
                c1 = (iota + (d + 1)) & (OUT - 1)
                c2 = (iota + (d + 2)) & (OUT - 1)
                c3 = (iota + (d + 3)) & (OUT - 1)
                a0 = a0 + plsc.load_gather(ra, [rowi, c0]) * plsc.load_gather(rb, [rowi, c0])
                a1 = a1 + plsc.load_gather(ra, [rowi, c1]) * plsc.load_gather(rb, [rowi, c1])
                a2 = a2 + plsc.load_gather(ra, [rowi, c2]) * plsc.load_gather(rb, [rowi, c2])
                a3 = a3 + plsc.load_gather(ra, [rowi, c3]) * plsc.load_gather(rb, [rowi, c3])
            ob[pl.ds(g * L, L)] = (a0 + a1) + (a2 + a3)
            return gcarry

        lax.fori_loop(0, CB // L, group, 0)

    gather(0, ra0, rb0, sem0)

    def body(j, carry):
        gwait(j, ra0, rb0, sem0)
        gather(j + 1, ra1, rb1, sem1)

        @pl.when(j >= 2)
        def _():
            owait(j - 2, ob0, osem0)

        compute(j, ra0, rb0, ob0)
        owrite(j, ob0, osem0)
        gwait(j + 1, ra1, rb1, sem1)

        @pl.when(j + 2 < KB2)
        def _():
            gather(j + 2, ra0, rb0, sem0)

        @pl.when(j >= 2)
        def _():
            owait(j - 1, ob1, osem1)

        compute(j + 1, ra1, rb1, ob1)
        owrite(j + 1, ob1, osem1)
        return carry

    lax.fori_loop(0, KB2 // 2, lambda t, cc: body(t * 2, cc), 0)
    owait(KB2 - 2, ob0, osem0)
    owait(KB2 - 1, ob1, osem1)


def _dinv_block(degp_blk, block_i):
    deg = jnp.sum(degp_blk, axis=0)[:, None] + 1.0
    rows = lax.broadcasted_iota(jnp.int32, (BN, 1), 0) + block_i * BN
    return jnp.where(rows < N, lax.rsqrt(deg), 0.0)


def _tc1_body(x_ref, w_ref, degp_ref, h1p_ref):
    i = pl.program_id(0)
    dinv = _dinv_block(degp_ref[...], i)
    h = jnp.dot(x_ref[...], w_ref[...], preferred_element_type=jnp.float32)
    h1p_ref[...] = h * dinv


def _tc2_body(p_ref, h1p_ref, degp_ref, w2_ref, b1_ref, h2p_ref):
    i = pl.program_id(0)
    dinv = _dinv_block(degp_ref[...], i)
    agg = p_ref[0] + p_ref[1] + h1p_ref[...]
    z1 = jnp.maximum(dinv * agg + b1_ref[...], 0.0)
    h2p_ref[...] = jnp.dot(z1, w2_ref[...], preferred_element_type=jnp.float32) * dinv


def _tc3_body(q_ref, h2p_ref, degp_ref, b2_ref, lat_ref):
    i = pl.program_id(0)
    dinv = _dinv_block(degp_ref[...], i)
    lat_ref[...] = dinv * (q_ref[0] + q_ref[1] + h2p_ref[...]) + b2_ref[...]


def _tc1(x_p, W1, degp):
    return pl.pallas_call(
        _tc1_body,
        grid=(NP // BN,),
        in_specs=[
            pl.BlockSpec((BN, DF), lambda i: (i, 0)),
            pl.BlockSpec((DF, H), lambda i: (0, 0)),
            pl.BlockSpec((NW, BN), lambda i: (0, i)),
        ],
        out_specs=pl.BlockSpec((BN, H), lambda i: (i, 0)),
        out_shape=jax.ShapeDtypeStruct((NP, H), jnp.float32),
    )(x_p, W1, degp)


def _tc2(p, h1p, degp, W2, b1):
    return pl.pallas_call(
        _tc2_body,
        grid=(NP // BN,),
        in_specs=[
            pl.BlockSpec((NC, BN, H), lambda i: (0, i, 0)),
            pl.BlockSpec((BN, H), lambda i: (i, 0)),
            pl.BlockSpec((NW, BN), lambda i: (0, i)),
            pl.BlockSpec((H, OUT), lambda i: (0, 0)),
            pl.BlockSpec((1, H), lambda i: (0, 0)),
        ],
        out_specs=pl.BlockSpec((BN, OUT), lambda i: (i, 0)),
        out_shape=jax.ShapeDtypeStruct((NP, OUT), jnp.float32),
    )(p, h1p, degp, W2, b1)


def _tc3(q, h2p, degp, b2):
    return pl.pallas_call(
        _tc3_body,
        grid=(NP // BN,),
        in_specs=[
            pl.BlockSpec((NC, BN, OUT), lambda i: (0, i, 0)),
            pl.BlockSpec((BN, OUT), lambda i: (i, 0)),
            pl.BlockSpec((NW, BN), lambda i: (0, i)),
            pl.BlockSpec((1, OUT), lambda i: (0, 0)),
        ],
        out_specs=pl.BlockSpec((BN, OUT), lambda i: (i, 0)),
        out_shape=jax.ShapeDtypeStruct((NP, OUT), jnp.float32),
    )(q, h2p, degp, b2)


def _pad_idx(idx, kch, cb):
    tot = NW * kch * cb
    p = jnp.full((tot,), PAD_NODE, jnp.int32).at[: idx.shape[0]].set(idx)
    return p.reshape(NW, kch, cb)


def kernel(x, edge_index, neg_edge, W1, b1, W2, b2):
    src1 = _pad_idx(edge_index[0], KCH1, CB1)
    dst1 = _pad_idx(edge_index[1], KCH1, CB1)
    src2 = _pad_idx(edge_index[0], KCH, CB)
    dst2 = _pad_idx(edge_index[1], KCH, CB)

    x_p = jnp.zeros((NP, DF), jnp.float32).at[:N].set(x)
    zeros_h = jnp.zeros((STRIPE, H), jnp.float32)
    zeros_o = jnp.zeros((STRIPE, OUT), jnp.float32)

    degp = _deg_kernel(dst2)
    h1p = _tc1(x_p, W1, degp)
    p1 = _agg_h(h1p, src1, dst1, zeros_h)
    h2p = _tc2(p1, h1p, degp, W2, b1.reshape(1, H))
    p2 = _agg_out(h2p, src2, dst2, zeros_o)
    latent = _tc3(p2, h2p, degp, b2.reshape(1, OUT))

    ia = _pad_idx(jnp.concatenate([edge_index[0], neg_edge[0]]), KB2, CB)
    ib = _pad_idx(jnp.concatenate([edge_index[1], neg_edge[1]]), KB2, CB)
    logits = _dec_kernel(latent, ia, ib)
    return logits.reshape(-1)[:E2]


# speedup vs baseline: 20.3513x; 1.3787x over previous
import functools

import jax
import jax.numpy as jnp
from jax import lax
from jax.experimental import pallas as pl
from jax.experimental.pallas import tpu as pltpu
from jax.experimental.pallas import tpu_sc as plsc

N = 10000
DF = 128
H = 128
OUT = 64
E = 320000
E2 = 2 * E

NC = 2
NS = 16
NW = NC * NS
L = 16

NP = 10240
STRIPE = NP // NS
PAD_NODE = NP - 1

CB = 128
KCH = 80
KB2 = 158

BN = 1024

_mesh = plsc.VectorSubcoreMesh(core_axis_name="c", subcore_axis_name="s")
_sc_params = pltpu.CompilerParams(
    needs_layout_passes=False, use_tc_tiling_on_sc=False
)


@functools.partial(
    pl.kernel,
    out_type=jax.ShapeDtypeStruct((NW, NP), jnp.float32),
    mesh=_mesh,
    scratch_types=[
        pltpu.VMEM((KCH, CB), jnp.int32),
        pltpu.VMEM((NP,), jnp.float32),
    ],
    compiler_params=_sc_params,
)
def _deg_kernel(dst_hbm, degp_hbm, idx_v, deg_v):
    c = lax.axis_index("c")
    s = lax.axis_index("s")
    wid = s * NC + c
    pltpu.sync_copy(dst_hbm.at[wid], idx_v)
    zero = jnp.zeros((L,), jnp.float32)

    def zbody(i, carry):
        deg_v[pl.ds(i * L, L)] = zero
        return carry

    lax.fori_loop(0, NP // L, zbody, 0)
    ones = jnp.ones((L,), jnp.float32)

    def cbody(j, carry):
        for g in range(CB // L):
            idx16 = idx_v[j, pl.ds(g * L, L)]
            plsc.addupdate_scatter(deg_v, [idx16], ones)
        return carry

    lax.fori_loop(0, KCH, cbody, 0)
    pltpu.sync_copy(deg_v, degp_hbm.at[wid])


def _make_agg(dim, kch, cb, table_in_spmem):
    scratch = [
        pltpu.VMEM((kch, cb), jnp.int32),
        pltpu.VMEM((kch, cb), jnp.int32),
        pltpu.VMEM((cb, dim), jnp.float32),
        pltpu.VMEM((cb, dim), jnp.float32),
        pltpu.VMEM_SHARED((NP, dim), jnp.float32),
        pltpu.SemaphoreType.DMA,
        pltpu.SemaphoreType.DMA,
    ]
    if table_in_spmem:
        scratch.append(pltpu.VMEM_SHARED((NP, dim), jnp.float32))

    @functools.partial(
        pl.kernel,
        out_type=jax.ShapeDtypeStruct((NC, NP, dim), jnp.float32),
        mesh=_mesh,
        scratch_types=scratch,
        name=f"edge_agg_{dim}",
        compiler_params=_sc_params,
    )
    def agg(table_hbm, src_hbm, dst_hbm, zeros_hbm, out_hbm, src_v, dst_v,
            rows0, rows1, shared, sem0, sem1, *rest):
        c = lax.axis_index("c")
        s = lax.axis_index("s")
        wid = s * NC + c
        stripe = pl.ds(s * STRIPE, STRIPE)
        pltpu.sync_copy(zeros_hbm, shared.at[stripe])
        pltpu.sync_copy(src_hbm.at[wid], src_v)
        pltpu.sync_copy(dst_hbm.at[wid], dst_v)
        if table_in_spmem:
            table = rest[0]
            pltpu.sync_copy(table_hbm.at[stripe], table.at[stripe])
        else:
            table = table_hbm
        plsc.subcore_barrier()

        def gather(j, rows, sem):
            pltpu.async_copy(table.at[src_v.at[j]], rows, sem)

        def gwait(j, rows, sem):
            pltpu.make_async_copy(table.at[src_v.at[j]], rows, sem).wait()

        gather(0, rows0, sem0)

        def body(j, carry):
            gwait(j, rows0, sem0)
            gather(j + 1, rows1, sem1)
            pltpu.sync_copy(rows0, shared.at[dst_v.at[j]], add=True)
            gwait(j + 1, rows1, sem1)

            @pl.when(j + 2 < kch)
            def _():
                gather(j + 2, rows0, sem0)

            pltpu.sync_copy(rows1, shared.at[dst_v.at[j + 1]], add=True)
            return carry

        lax.fori_loop(0, kch // 2, lambda t, cc: body(t * 2, cc), 0)
        plsc.subcore_barrier()
        pltpu.sync_copy(shared.at[stripe], out_hbm.at[c, stripe])

    return agg


_agg_out = _make_agg(OUT, KCH, CB, True)


@functools.partial(
    pl.kernel,
    out_type=jax.ShapeDtypeStruct((NW, KB2 * CB), jnp.float32),
    mesh=_mesh,
    scratch_types=[
        pltpu.VMEM((KB2, CB), jnp.int32),
        pltpu.VMEM((KB2, CB), jnp.int32),
        pltpu.VMEM((CB, OUT), jnp.float32),
        pltpu.VMEM((CB, OUT), jnp.float32),
        pltpu.VMEM((CB, OUT), jnp.float32),
        pltpu.VMEM((CB, OUT), jnp.float32),
        pltpu.VMEM((CB,), jnp.float32),
        pltpu.VMEM((CB,), jnp.float32),
        pltpu.VMEM_SHARED((NP, OUT), jnp.float32),
        pltpu.SemaphoreType.DMA,
        pltpu.SemaphoreType.DMA,
        pltpu.SemaphoreType.DMA,
        pltpu.SemaphoreType.DMA,
    ],
    name="edge_decoder",
    compiler_params=_sc_params,
)
def _dec_kernel(latent_hbm, ia_hbm, ib_hbm, out_hbm, ia_v, ib_v,
                ra0, rb0, ra1, rb1, ob0, ob1, table,
                sem0, sem1, osem0, osem1):
    c = lax.axis_index("c")
    s = lax.axis_index("s")
    wid = s * NC + c
    stripe = pl.ds(s * STRIPE, STRIPE)
    pltpu.sync_copy(latent_hbm.at[stripe], table.at[stripe])
    pltpu.sync_copy(ia_hbm.at[wid], ia_v)
    pltpu.sync_copy(ib_hbm.at[wid], ib_v)
    plsc.subcore_barrier()
    iota = lax.iota(jnp.int32, L)

    def gather(j, ra, rb, sem):
        pltpu.async_copy(table.at[ia_v.at[j]], ra, sem)
        pltpu.async_copy(table.at[ib_v.at[j]], rb, sem)

    def gwait(j, ra, rb, sem):
        pltpu.make_async_copy(table.at[ia_v.at[j]], ra, sem).wait()
        pltpu.make_async_copy(table.at[ib_v.at[j]], rb, sem).wait()

    def owrite(j, ob, osem):
        pltpu.async_copy(ob, out_hbm.at[wid, pl.ds(j * CB, CB)], osem)

    def owait(j, ob, osem):
        pltpu.make_async_copy(
            ob, out_hbm.at[wid, pl.ds(j * CB, CB)], osem
        ).wait()

    def compute(j, ra, rb, ob):
        def group(g, gcarry):
            rowi = g * L + iota
            a0 = jnp.zeros((L,), jnp.float32)
            a1 = a0
            a2 = a0
            a3 = a0
            for d in range(0, OUT, 4):
                c0 = (iota + d) & (OUT - 1)
                c1 = (iota + (d + 1)) & (OUT - 1)
                c2 = (iota + (d + 2)) & (OUT - 1)
                c3 = (iota + (d + 3)) & (OUT - 1)
                a0 = a0 + plsc.load_gather(ra, [rowi, c0]) * plsc.load_gather(rb, [rowi, c0])
                a1 = a1 + plsc.load_gather(ra, [rowi, c1]) * plsc.load_gather(rb, [rowi, c1])
                a2 = a2 + plsc.load_gather(ra, [rowi, c2]) * plsc.load_gather(rb, [rowi, c2])
                a3 = a3 + plsc.load_gather(ra, [rowi, c3]) * plsc.load_gather(rb, [rowi, c3])
            ob[pl.ds(g * L, L)] = (a0 + a1) + (a2 + a3)
            return gcarry

        lax.fori_loop(0, CB // L, group, 0)

    gather(0, ra0, rb0, sem0)

    def body(j, carry):
        gwait(j, ra0, rb0, sem0)
        gather(j + 1, ra1, rb1, sem1)

        @pl.when(j >= 2)
        def _():
            owait(j - 2, ob0, osem0)

        compute(j, ra0, rb0, ob0)
        owrite(j, ob0, osem0)
        gwait(j + 1, ra1, rb1, sem1)

        @pl.when(j + 2 < KB2)
        def _():
            gather(j + 2, ra0, rb0, sem0)

        @pl.when(j >= 2)
        def _():
            owait(j - 1, ob1, osem1)

        compute(j + 1, ra1, rb1, ob1)
        owrite(j + 1, ob1, osem1)
        return carry

    lax.fori_loop(0, KB2 // 2, lambda t, cc: body(t * 2, cc), 0)
    owait(KB2 - 2, ob0, osem0)
    owait(KB2 - 1, ob1, osem1)


def _dinv_block(degp_blk, block_i):
    deg = jnp.sum(degp_blk, axis=0)[:, None] + 1.0
    rows = lax.broadcasted_iota(jnp.int32, (BN, 1), 0) + block_i * BN
    return jnp.where(rows < N, lax.rsqrt(deg), 0.0)


def _tc1_body(x_ref, w_ref, degp_ref, h1p_ref):
    i = pl.program_id(0)
    dinv = _dinv_block(degp_ref[...], i)
    h = jnp.dot(x_ref[...], w_ref[...], preferred_element_type=jnp.float32)
    h1p_ref[0, ...] = h[:, :OUT] * dinv
    h1p_ref[1, ...] = h[:, OUT:] * dinv


def _tc2_body(pa_ref, pb_ref, h1p_ref, degp_ref, w2_ref, b1_ref, h2p_ref):
    i = pl.program_id(0)
    dinv = _dinv_block(degp_ref[...], i)
    agg0 = pa_ref[0] + pa_ref[1] + h1p_ref[0]
    agg1 = pb_ref[0] + pb_ref[1] + h1p_ref[1]
    z10 = jnp.maximum(dinv * agg0 + b1_ref[:, :OUT], 0.0)
    z11 = jnp.maximum(dinv * agg1 + b1_ref[:, OUT:], 0.0)
    h2 = jnp.dot(z10, w2_ref[:OUT], preferred_element_type=jnp.float32)
    h2 = h2 + jnp.dot(z11, w2_ref[OUT:], preferred_element_type=jnp.float32)
    h2p_ref[...] = h2 * dinv


def _tc3_body(q_ref, h2p_ref, degp_ref, b2_ref, lat_ref):
    i = pl.program_id(0)
    dinv = _dinv_block(degp_ref[...], i)
    lat_ref[...] = dinv * (q_ref[0] + q_ref[1] + h2p_ref[...]) + b2_ref[...]


def _tc1(x_p, W1, degp):
    return pl.pallas_call(
        _tc1_body,
        grid=(NP // BN,),
        in_specs=[
            pl.BlockSpec((BN, DF), lambda i: (i, 0)),
            pl.BlockSpec((DF, H), lambda i: (0, 0)),
            pl.BlockSpec((NW, BN), lambda i: (0, i)),
        ],
        out_specs=pl.BlockSpec((2, BN, OUT), lambda i: (0, i, 0)),
        out_shape=jax.ShapeDtypeStruct((2, NP, OUT), jnp.float32),
    )(x_p, W1, degp)


def _tc2(pa, pb, h1p, degp, W2, b1):
    return pl.pallas_call(
        _tc2_body,
        grid=(NP // BN,),
        in_specs=[
            pl.BlockSpec((NC, BN, OUT), lambda i: (0, i, 0)),
            pl.BlockSpec((NC, BN, OUT), lambda i: (0, i, 0)),
            pl.BlockSpec((2, BN, OUT), lambda i: (0, i, 0)),
            pl.BlockSpec((NW, BN), lambda i: (0, i)),
            pl.BlockSpec((H, OUT), lambda i: (0, 0)),
            pl.BlockSpec((1, H), lambda i: (0, 0)),
        ],
        out_specs=pl.BlockSpec((BN, OUT), lambda i: (i, 0)),
        out_shape=jax.ShapeDtypeStruct((NP, OUT), jnp.float32),
    )(pa, pb, h1p, degp, W2, b1)


def _tc3(q, h2p, degp, b2):
    return pl.pallas_call(
        _tc3_body,
        grid=(NP // BN,),
        in_specs=[
            pl.BlockSpec((NC, BN, OUT), lambda i: (0, i, 0)),
            pl.BlockSpec((BN, OUT), lambda i: (i, 0)),
            pl.BlockSpec((NW, BN), lambda i: (0, i)),
            pl.BlockSpec((1, OUT), lambda i: (0, 0)),
        ],
        out_specs=pl.BlockSpec((BN, OUT), lambda i: (i, 0)),
        out_shape=jax.ShapeDtypeStruct((NP, OUT), jnp.float32),
    )(q, h2p, degp, b2)


def _pad_idx(idx, kch, cb):
    tot = NW * kch * cb
    p = jnp.full((tot,), PAD_NODE, jnp.int32).at[: idx.shape[0]].set(idx)
    return p.reshape(NW, kch, cb)


def kernel(x, edge_index, neg_edge, W1, b1, W2, b2):
    src2 = _pad_idx(edge_index[0], KCH, CB)
    dst2 = _pad_idx(edge_index[1], KCH, CB)

    x_p = jnp.zeros((NP, DF), jnp.float32).at[:N].set(x)
    zeros_o = jnp.zeros((STRIPE, OUT), jnp.float32)

    degp = _deg_kernel(dst2)
    h1p = _tc1(x_p, W1, degp)
    p1a = _agg_out(h1p[0], src2, dst2, zeros_o)
    p1b = _agg_out(h1p[1], src2, dst2, zeros_o)
    h2p = _tc2(p1a, p1b, h1p, degp, W2, b1.reshape(1, H))
    p2 = _agg_out(h2p, src2, dst2, zeros_o)
    latent = _tc3(p2, h2p, degp, b2.reshape(1, OUT))

    ia = _pad_idx(jnp.concatenate([edge_index[0], neg_edge[0]]), KB2, CB)
    ib = _pad_idx(jnp.concatenate([edge_index[1], neg_edge[1]]), KB2, CB)
    logits = _dec_kernel(latent, ia, ib)
    return logits.reshape(-1)[:E2]
